# Initial kernel scaffold; baseline (speedup 1.0000x reference)
import jax
import jax.numpy as jnp
from jax.experimental import pallas as pl


def kernel(inputs, adj, mask_item, item, input_times, adj_all, num, embedding, a_0, a_1, a_2, a_3, w_time, w1g, w2g, w3g, bg):
    raise NotImplementedError("write your pallas kernel here")


# speedup vs baseline: 3.3917x; 3.3917x over previous
import functools

import jax
import jax.numpy as jnp
from jax import lax
from jax.experimental import pallas as pl
from jax.experimental.pallas import tpu as pltpu
from jax.experimental.pallas import tpu_sc as plsc

D = 128
S = 12
BATCH = 1024
SEQ = 20
BL = BATCH * SEQ
SLOPE = 0.2

NC = 2
NW = 32

BB = 16
GRID = BATCH // BB


CG = 320
WROWS = BL // NW
NCH_N = (BL * S) // (NW * CG)
SPAN = BL // CG


def _gather_hi_body(in_ids, item_ids, emb_hbm, h_out, item_out,
                    idx0, idx1, buf0, buf1, sem0, sem1):
    wid = lax.axis_index("s") * NC + lax.axis_index("c")
    base = wid * WROWS
    idx = (idx0, idx1)
    buf = (buf0, buf1)
    sem = (sem0, sem1)

    def load(ids_src, off, p):
        pltpu.sync_copy(ids_src.at[pl.ds(off, CG)], idx[p])
        return pltpu.async_copy(emb_hbm.at[idx[p]], buf[p], sem[p])

    g0 = load(in_ids, base, 0)
    g1 = load(in_ids, base + CG, 1)
    g0.wait()
    pltpu.sync_copy(buf0, h_out.at[pl.ds(base, CG), :])
    g0 = load(item_ids, base, 0)
    g1.wait()
    pltpu.sync_copy(buf1, h_out.at[pl.ds(base + CG, CG), :])
    g1 = load(item_ids, base + CG, 1)
    g0.wait()
    pltpu.sync_copy(buf0, item_out.at[pl.ds(base, CG), :])
    g1.wait()
    pltpu.sync_copy(buf1, item_out.at[pl.ds(base + CG, CG), :])


def _gather_nbr_body(nbr_ids, emb_hbm, neigh_out,
                     idx0, idx1, buf0, buf1, sem0, sem1):
    wid = lax.axis_index("s") * NC + lax.axis_index("c")
    idx = (idx0, idx1)
    buf = (buf0, buf1)
    sem = (sem0, sem1)
    nbase = wid * NCH_N

    def start(g, p):
        pltpu.sync_copy(nbr_ids.at[pl.ds(g * CG, CG)], idx[p])
        return pltpu.async_copy(emb_hbm.at[idx[p]], buf[p], sem[p])

    def drain(p):
        pltpu.make_async_copy(emb_hbm.at[idx[p]], buf[p], sem[p]).wait()

    def write(g, p):
        s = g // SPAN
        n0 = (g % SPAN) * CG
        pltpu.sync_copy(buf[p], neigh_out.at[s, pl.ds(n0, CG), :])

    start(nbase, 0)

    def pair(i, carry):
        e = nbase + 2 * i
        start(e + 1, 1)
        drain(0)
        write(e, 0)
        start(e + 2, 0)
        drain(1)
        write(e + 1, 1)
        return carry

    lax.fori_loop(0, NCH_N // 2 - 1, pair, 0)
    e = nbase + NCH_N - 2
    start(e + 1, 1)
    drain(0)
    write(e, 0)
    drain(1)
    write(e + 1, 1)


_SC_SCRATCH = [
    pltpu.VMEM((CG,), jnp.int32),
    pltpu.VMEM((CG,), jnp.int32),
    pltpu.VMEM((CG, D), jnp.float32),
    pltpu.VMEM((CG, D), jnp.float32),
    pltpu.SemaphoreType.DMA,
    pltpu.SemaphoreType.DMA,
]


@functools.lru_cache(maxsize=None)
def _make_sc_gathers(interpret=False):
    mesh = plsc.VectorSubcoreMesh(core_axis_name="c", subcore_axis_name="s")
    ghi = pl.kernel(
        _gather_hi_body,
        out_type=[
            jax.ShapeDtypeStruct((BL, D), jnp.float32),
            jax.ShapeDtypeStruct((BL, D), jnp.float32),
        ],
        mesh=mesh,
        scratch_types=list(_SC_SCRATCH),
        interpret=interpret,
    )
    gn = pl.kernel(
        _gather_nbr_body,
        out_type=jax.ShapeDtypeStruct((S, BL, D), jnp.float32),
        mesh=mesh,
        scratch_types=list(_SC_SCRATCH),
        interpret=interpret,
    )
    return ghi, gn


def _leaky(x, slope):
    return jnp.where(x >= 0, x, slope * x)


def _local_body(h_ref, item_ref, adj_ref, tcol_ref, trow_ref, mask_ref,
                a_ref, wt_ref, hl_out, sess_out):
    N = BB * SEQ
    h = h_ref[...]
    item = item_ref[...]
    A = a_ref[...]
    wt = wt_ref[0, 0]
    mask = mask_ref[...]

    dot = functools.partial(lax.dot_general,
                            preferred_element_type=jnp.float32)

    i0 = lax.broadcasted_iota(jnp.int32, (BB, N), 0)
    i1 = lax.broadcasted_iota(jnp.int32, (BB, N), 1)
    seg_r = (i1 // SEQ == i0).astype(jnp.float32)
    sess = dot(seg_r, item * mask, (((1,), (0,)), ((), ())))
    msum = dot(seg_r, mask, (((1,), (0,)), ((), ())))
    sess_out[...] = sess / msum

    h3 = h.reshape(BB, SEQ, D)
    adj3 = adj_ref[...]
    bdot = lambda x, y: dot(x, y, (((2,), (2,)), ((0,), (0,))))
    e0 = _leaky(bdot(h3 * A[0:1, :][None], h3), SLOPE)
    e1 = _leaky(bdot(h3 * A[1:2, :][None], h3), SLOPE)
    e2 = _leaky(bdot(h3 * A[2:3, :][None], h3), SLOPE)
    e3 = _leaky(bdot(h3 * A[3:4, :][None], h3), SLOPE)
    att = jnp.full((BB, SEQ, SEQ), -9e15, jnp.float32)
    att = jnp.where(adj3 == 1, e0, att)
    att = jnp.where(adj3 == 2, e1, att)
    att = jnp.where(adj3 == 3, e2, att)
    att = jnp.where(adj3 == 4, e3, att)
    tdiff = jnp.abs(tcol_ref[...] - trow_ref[...])
    att = att - wt * tdiff
    att = att - jnp.max(att, axis=-1, keepdims=True)
    att = jnp.exp(att)
    att = att / jnp.sum(att, axis=-1, keepdims=True)
    hl_out[...] = dot(att, h3, (((2,), (1,)), ((0,), (0,))))


def _global_body(h_ref, neigh_ref, wq_ref, sess_ref, m_ref, r_ref, w2_ref,
                 w3a_ref, w3b_ref, bg_ref, hg_out):
    N = BB * SEQ
    h = h_ref[...]
    M = m_ref[...]
    r = r_ref[...]
    w2 = w2_ref[...]
    w3a = w3a_ref[...]
    w3b = w3b_ref[...]
    bgr = bg_ref[...]

    dot = functools.partial(lax.dot_general,
                            preferred_element_type=jnp.float32)

    j0 = lax.broadcasted_iota(jnp.int32, (N, BB), 0)
    j1 = lax.broadcasted_iota(jnp.int32, (N, BB), 1)
    seg_e = (j0 // SEQ == j1).astype(jnp.float32)
    extra = dot(seg_e, sess_ref[...], (((1,), (0,)), ((), ())))

    nc = jnp.concatenate([neigh_ref[s] for s in range(S)], axis=0)
    wqc = jnp.concatenate([wq_ref[s] for s in range(S)], axis=0)
    extra_rep = jnp.concatenate([extra] * S, axis=0)
    ts = dot(nc * extra_rep, M, (((1,), (0,)), ((), ())))
    ts = _leaky(ts + wqc * r, 0.2)
    esc = jnp.exp(dot(ts, w2, (((1,), (0,)), ((), ()))))
    esum = jnp.zeros((N, 1), jnp.float32)
    for s in range(S):
        esum = esum + esc[s * N:(s + 1) * N, :]
    wn = esc * nc
    agg = jnp.zeros((N, D), jnp.float32)
    for s in range(S):
        agg = agg + wn[s * N:(s + 1) * N, :]
    agg = agg / esum
    hg = dot(h, w3a, (((1,), (0,)), ((), ()))) + \
        dot(agg, w3b, (((1,), (0,)), ((), ()))) + bgr
    hg_out[...] = jnp.maximum(hg, 0.0)


def _make_local(interpret=False):
    row_spec = lambda rows: pl.BlockSpec((rows, D), lambda i: (i, 0))
    full2 = lambda shape: pl.BlockSpec(shape, lambda i: (0,) * len(shape))
    return pl.pallas_call(
        _local_body,
        grid=(GRID,),
        in_specs=[
            row_spec(BB * SEQ),
            row_spec(BB * SEQ),
            pl.BlockSpec((BB, SEQ, SEQ), lambda i: (i, 0, 0)),
            pl.BlockSpec((BB, SEQ, 1), lambda i: (i, 0, 0)),
            pl.BlockSpec((BB, 1, SEQ), lambda i: (i, 0, 0)),
            pl.BlockSpec((BB * SEQ, 1), lambda i: (i, 0)),
            full2((4, D)),
            full2((1, 1)),
        ],
        out_specs=[pl.BlockSpec((BB, SEQ, D), lambda i: (i, 0, 0)),
                   pl.BlockSpec((BB, D), lambda i: (i, 0))],
        out_shape=[
            jax.ShapeDtypeStruct((BATCH, SEQ, D), jnp.float32),
            jax.ShapeDtypeStruct((BATCH, D), jnp.float32),
        ],
        interpret=interpret,
    )


def _make_global(interpret=False):
    row_spec = lambda rows: pl.BlockSpec((rows, D), lambda i: (i, 0))
    full2 = lambda shape: pl.BlockSpec(shape, lambda i: (0,) * len(shape))
    return pl.pallas_call(
        _global_body,
        grid=(GRID,),
        in_specs=[
            row_spec(BB * SEQ),
            pl.BlockSpec((S, BB * SEQ, D), lambda i: (0, i, 0)),
            pl.BlockSpec((S, BB * SEQ, 1), lambda i: (0, i, 0)),
            pl.BlockSpec((BB, D), lambda i: (i, 0)),
            full2((D, D)),
            full2((1, D)),
            full2((D, 1)),
            full2((D, D)),
            full2((D, D)),
            full2((1, D)),
        ],
        out_specs=[row_spec(BB * SEQ)],
        out_shape=[jax.ShapeDtypeStruct((BL, D), jnp.float32)],
        interpret=interpret,
    )


_LOCAL = _make_local()
_GLOBAL = _make_global()


def kernel(inputs, adj, mask_item, item, input_times, adj_all, num, embedding,
           a_0, a_1, a_2, a_3, w_time, w1g, w2g, w3g, bg):
    inputs_flat = inputs.reshape(-1).astype(jnp.int32)
    item_flat = item.reshape(-1).astype(jnp.int32)
    adj_all = adj_all.astype(jnp.int32)

    nbr_ids = adj_all[inputs_flat]
    weight = num[inputs_flat]
    gather_hi, gather_nbr = _make_sc_gathers()
    h_flat, item_emb = gather_hi(inputs_flat, item_flat, embedding)
    neigh3 = gather_nbr(nbr_ids.T.reshape(-1), embedding)

    amat = jnp.concatenate([a_0, a_1, a_2, a_3], axis=1).T
    hl, sess = _LOCAL(
        h_flat, item_emb,
        adj.astype(jnp.int32),
        input_times.reshape(BATCH, SEQ, 1),
        input_times.reshape(BATCH, 1, SEQ),
        mask_item.reshape(-1, 1),
        amat, w_time.reshape(1, 1),
    )
    (hg,) = _GLOBAL(
        h_flat, neigh3, weight.T.reshape(S, BL, 1), sess,
        w1g[:D, :], w1g[D:D + 1, :], w2g,
        w3g[:D, :], w3g[D:, :], bg.reshape(1, D),
    )
    return hl, hg.reshape(BATCH, SEQ, D)


# speedup vs baseline: 3.6581x; 1.0786x over previous
import functools

import jax
import jax.numpy as jnp
from jax import lax
from jax.experimental import pallas as pl
from jax.experimental.pallas import tpu as pltpu
from jax.experimental.pallas import tpu_sc as plsc

D = 128
S = 12
BATCH = 1024
SEQ = 20
BL = BATCH * SEQ
SLOPE = 0.2

NC = 2
NW = 32

BB = 16
GRID = BATCH // BB


CG = 320
WROWS = BL // NW
NCH_N = (BL * S) // (NW * CG)
LANES = 16
SPAD = 16


def _gather_body(in_ids, item_ids, nbr_t, emb_hbm,
                 h_out, item_out, neigh_out,
                 idx0, idx1, buf0, buf1, sem0, sem1):
    wid = lax.axis_index("s") * NC + lax.axis_index("c")
    base = wid * WROWS
    idx = (idx0, idx1)
    buf = (buf0, buf1)
    sem = (sem0, sem1)

    def load(ids_src, off, p):
        pltpu.sync_copy(ids_src.at[pl.ds(off, CG)], idx[p])
        return pltpu.async_copy(emb_hbm.at[idx[p]], buf[p], sem[p])

    g0 = load(in_ids, base, 0)
    g1 = load(in_ids, base + CG, 1)
    g0.wait()
    pltpu.sync_copy(buf0, h_out.at[pl.ds(base, CG), :])
    g0 = load(item_ids, base, 0)
    g1.wait()
    pltpu.sync_copy(buf1, h_out.at[pl.ds(base + CG, CG), :])
    g1 = load(item_ids, base + CG, 1)
    g0.wait()
    pltpu.sync_copy(buf0, item_out.at[pl.ds(base, CG), :])
    g1.wait()
    pltpu.sync_copy(buf1, item_out.at[pl.ds(base + CG, CG), :])

    def start(c, p):
        s_i = c // 2
        half = c % 2
        off = s_i * BL + base + half * CG
        pltpu.sync_copy(nbr_t.at[pl.ds(off, CG)], idx[p])
        return pltpu.async_copy(emb_hbm.at[idx[p]], buf[p], sem[p])

    def drain(p):
        pltpu.make_async_copy(emb_hbm.at[idx[p]], buf[p], sem[p]).wait()

    def write(c, p):
        s_i = c // 2
        half = c % 2
        pltpu.sync_copy(
            buf[p], neigh_out.at[s_i, pl.ds(base + half * CG, CG), :])

    start(0, 0)

    def pair(i, carry):
        e = 2 * i
        start(e + 1, 1)
        drain(0)
        write(e, 0)
        start(e + 2, 0)
        drain(1)
        write(e + 1, 1)
        return carry

    lax.fori_loop(0, NCH_N // 2 - 1, pair, 0)
    e = NCH_N - 2
    start(e + 1, 1)
    drain(0)
    write(e, 0)
    drain(1)
    write(e + 1, 1)


@functools.lru_cache(maxsize=None)
def _make_sc_gather(interpret=False):
    mesh = plsc.VectorSubcoreMesh(core_axis_name="c", subcore_axis_name="s")
    return pl.kernel(
        _gather_body,
        out_type=[
            jax.ShapeDtypeStruct((BL, D), jnp.float32),
            jax.ShapeDtypeStruct((BL, D), jnp.float32),
            jax.ShapeDtypeStruct((S, BL, D), jnp.float32),
        ],
        mesh=mesh,
        scratch_types=[
            pltpu.VMEM((CG,), jnp.int32),
            pltpu.VMEM((CG,), jnp.int32),
            pltpu.VMEM((CG, D), jnp.float32),
            pltpu.VMEM((CG, D), jnp.float32),
            pltpu.SemaphoreType.DMA,
            pltpu.SemaphoreType.DMA,
        ],
        interpret=interpret,
    )


def _leaky(x, slope):
    return jnp.where(x >= 0, x, slope * x)


def _dense_body(h_ref, item_ref, neigh_ref, wq_ref, adj_ref, tcol_ref, trow_ref,
                mask_ref, a_ref, wt_ref, m_ref, r_ref, w2_ref, w3a_ref, w3b_ref,
                bg_ref, hl_out, hg_out):
    N = BB * SEQ
    h = h_ref[...]
    item = item_ref[...]
    A = a_ref[...]
    wt = wt_ref[0, 0]
    M = m_ref[...]
    r = r_ref[...]
    w2 = w2_ref[...]
    w3a = w3a_ref[...]
    w3b = w3b_ref[...]
    bgr = bg_ref[...]
    mask = mask_ref[...]

    dot = functools.partial(lax.dot_general,
                            preferred_element_type=jnp.float32)

    i0 = lax.broadcasted_iota(jnp.int32, (BB, N), 0)
    i1 = lax.broadcasted_iota(jnp.int32, (BB, N), 1)
    seg_r = (i1 // SEQ == i0).astype(jnp.float32)
    j0 = lax.broadcasted_iota(jnp.int32, (N, BB), 0)
    j1 = lax.broadcasted_iota(jnp.int32, (N, BB), 1)
    seg_e = (j0 // SEQ == j1).astype(jnp.float32)
    sess = dot(seg_r, item * mask, (((1,), (0,)), ((), ())))
    msum = dot(seg_r, mask, (((1,), (0,)), ((), ())))
    sess = sess / msum
    extra = dot(seg_e, sess, (((1,), (0,)), ((), ())))

    h3 = h.reshape(BB, SEQ, D)
    adj3 = adj_ref[...]
    bdot = lambda x, y: dot(x, y, (((2,), (2,)), ((0,), (0,))))
    e0 = _leaky(bdot(h3 * A[0:1, :][None], h3), SLOPE)
    e1 = _leaky(bdot(h3 * A[1:2, :][None], h3), SLOPE)
    e2 = _leaky(bdot(h3 * A[2:3, :][None], h3), SLOPE)
    e3 = _leaky(bdot(h3 * A[3:4, :][None], h3), SLOPE)
    att = jnp.full((BB, SEQ, SEQ), -9e15, jnp.float32)
    att = jnp.where(adj3 == 1, e0, att)
    att = jnp.where(adj3 == 2, e1, att)
    att = jnp.where(adj3 == 3, e2, att)
    att = jnp.where(adj3 == 4, e3, att)
    tdiff = jnp.abs(tcol_ref[...] - trow_ref[...])
    att = att - wt * tdiff
    att = att - jnp.max(att, axis=-1, keepdims=True)
    att = jnp.exp(att)
    att = att / jnp.sum(att, axis=-1, keepdims=True)
    hl_out[...] = dot(att, h3, (((2,), (1,)), ((0,), (0,))))

    nc = jnp.concatenate([neigh_ref[s] for s in range(S)], axis=0)
    wqc = jnp.concatenate([wq_ref[s] for s in range(S)], axis=0)
    extra_rep = jnp.concatenate([extra] * S, axis=0)
    ts = dot(nc * extra_rep, M, (((1,), (0,)), ((), ())))
    ts = _leaky(ts + wqc * r, 0.2)
    esc = jnp.exp(dot(ts, w2, (((1,), (0,)), ((), ()))))
    esum = jnp.zeros((N, 1), jnp.float32)
    for s in range(S):
        esum = esum + esc[s * N:(s + 1) * N, :]
    wn = esc * nc
    agg = jnp.zeros((N, D), jnp.float32)
    for s in range(S):
        agg = agg + wn[s * N:(s + 1) * N, :]
    agg = agg / esum
    hg = dot(h, w3a, (((1,), (0,)), ((), ()))) + \
        dot(agg, w3b, (((1,), (0,)), ((), ()))) + bgr
    hg_out[...] = jnp.maximum(hg, 0.0)


def _make_dense(interpret=False):
    row_spec = lambda rows: pl.BlockSpec((rows, D), lambda i: (i, 0))
    full2 = lambda shape: pl.BlockSpec(shape, lambda i: (0,) * len(shape))
    return pl.pallas_call(
        _dense_body,
        grid=(GRID,),
        in_specs=[
            row_spec(BB * SEQ),
            row_spec(BB * SEQ),
            pl.BlockSpec((S, BB * SEQ, D), lambda i: (0, i, 0)),
            pl.BlockSpec((S, BB * SEQ, 1), lambda i: (0, i, 0)),
            pl.BlockSpec((BB, SEQ, SEQ), lambda i: (i, 0, 0)),
            pl.BlockSpec((BB, SEQ, 1), lambda i: (i, 0, 0)),
            pl.BlockSpec((BB, 1, SEQ), lambda i: (i, 0, 0)),
            pl.BlockSpec((BB * SEQ, 1), lambda i: (i, 0)),
            full2((4, D)),
            full2((1, 1)),
            full2((D, D)),
            full2((1, D)),
            full2((D, 1)),
            full2((D, D)),
            full2((D, D)),
            full2((1, D)),
        ],
        out_specs=[pl.BlockSpec((BB, SEQ, D), lambda i: (i, 0, 0)),
                   row_spec(BB * SEQ)],
        out_shape=[
            jax.ShapeDtypeStruct((BATCH, SEQ, D), jnp.float32),
            jax.ShapeDtypeStruct((BL, D), jnp.float32),
        ],
        interpret=interpret,
    )


_DENSE = _make_dense()


def kernel(inputs, adj, mask_item, item, input_times, adj_all, num, embedding,
           a_0, a_1, a_2, a_3, w_time, w1g, w2g, w3g, bg):
    inputs_flat = inputs.reshape(-1).astype(jnp.int32)
    item_flat = item.reshape(-1).astype(jnp.int32)
    adj_all = adj_all.astype(jnp.int32)

    dn = lax.GatherDimensionNumbers(
        offset_dims=(0,), collapsed_slice_dims=(0,), start_index_map=(0,))
    gt = functools.partial(
        lax.gather, dimension_numbers=dn, slice_sizes=(1, S),
        mode=lax.GatherScatterMode.PROMISE_IN_BOUNDS)
    nbr_t = gt(adj_all, inputs_flat[:, None])
    weight_t = gt(num, inputs_flat[:, None])
    h_flat, item_emb, neigh3 = _make_sc_gather()(
        inputs_flat, item_flat, nbr_t.reshape(-1), embedding)

    amat = jnp.concatenate([a_0, a_1, a_2, a_3], axis=1).T
    hl, hg = _DENSE(
        h_flat, item_emb, neigh3,
        weight_t.reshape(S, BL, 1),
        adj.astype(jnp.int32),
        input_times.reshape(BATCH, SEQ, 1),
        input_times.reshape(BATCH, 1, SEQ),
        mask_item.reshape(-1, 1),
        amat, w_time.reshape(1, 1),
        w1g[:D, :], w1g[D:D + 1, :], w2g,
        w3g[:D, :], w3g[D:, :], bg.reshape(1, D),
    )
    return hl, hg.reshape(BATCH, SEQ, D)
